# baseline (speedup 1.0000x reference)
import functools

import jax
import jax.numpy as jnp
from jax import lax
from jax.experimental import pallas as pl
from jax.experimental.pallas import tpu as pltpu
from jax.experimental.pallas import tpu_sc as plsc

BATCH = 4096
VOCAB = 8192
SEQ = 200
D = 64
NC = 2
NS = 16
NW = NC * NS
BPW = BATCH // NW
L = 16
NG = D // L
G0 = 128
G1 = SEQ - G0
R = 4

_mesh = plsc.VectorSubcoreMesh(
    core_axis_name="c", subcore_axis_name="s", num_cores=NC, num_subcores=NS
)


@functools.partial(
    pl.kernel,
    out_type=jax.ShapeDtypeStruct((BATCH, SEQ, D), jnp.float32),
    mesh=_mesh,
    scratch_types=[
        pltpu.VMEM((SEQ, D), jnp.float32),
        pltpu.VMEM((BPW,), jnp.int32),
        pltpu.VMEM((BPW, D), jnp.float32),
        pltpu.VMEM((BPW, SEQ), jnp.int32),
        pltpu.VMEM((R, SEQ, D), jnp.float32),
        pltpu.VMEM_SHARED((VOCAB, D), jnp.float32),
        pltpu.SemaphoreType.DMA,
        pltpu.SemaphoreType.DMA,
        pltpu.SemaphoreType.DMA,
        pltpu.SemaphoreType.DMA,
        pltpu.SemaphoreType.DMA,
        pltpu.SemaphoreType.DMA,
        pltpu.SemaphoreType.DMA,
        pltpu.SemaphoreType.DMA,
        pltpu.SemaphoreType.DMA,
    ],
    compiler_params=pltpu.CompilerParams(use_tc_tiling_on_sc=False),
)
def _embed(x_hbm, cond_hbm, lat_hbm, ct_hbm, pe_hbm, out_hbm,
           pe_v, ci_v, cr_v, idx_v, rows_v, lat_s,
           gsem0, gsem1, gsem2, gsem3, osem0, osem1, osem2, osem3, csem):
    wid = lax.axis_index("s") * NC + lax.axis_index("c")
    base = wid * BPW
    gsems = (gsem0, gsem1, gsem2, gsem3)
    osems = (osem0, osem1, osem2, osem3)

    def gather_descs(buf, j):
        d0 = pltpu.make_async_copy(
            lat_s.at[idx_v.at[j, pl.ds(0, G0)]],
            rows_v.at[buf, pl.ds(0, G0)], gsems[buf])
        d1 = pltpu.make_async_copy(
            lat_s.at[idx_v.at[j, pl.ds(G0, G1)]],
            rows_v.at[buf, pl.ds(G0, G1)], gsems[buf])
        return d0, d1

    def start_gather(buf, j):
        d0, d1 = gather_descs(buf, j)
        d0.start()
        d1.start()

    def wait_gather(buf, j):
        d0, d1 = gather_descs(buf, j)
        d0.wait()
        d1.wait()

    def out_desc(buf, j):
        return pltpu.make_async_copy(rows_v.at[buf], out_hbm.at[base + j], osems[buf])

    def compute(buf, j):
        c = [cr_v[j, pl.ds(g * L, L)] for g in range(NG)]

        @pl.loop(0, SEQ)
        def _(s):
            for g in range(NG):
                sl = pl.ds(g * L, L)
                rows_v[buf, s, sl] = rows_v[buf, s, sl] + pe_v[s, sl] + c[g]

    sid = lax.axis_index("s")
    rpt = VOCAB // NS
    pltpu.sync_copy(lat_hbm.at[pl.ds(sid * rpt, rpt)],
                    lat_s.at[pl.ds(sid * rpt, rpt)])
    plsc.subcore_barrier()

    pltpu.sync_copy(pe_hbm, pe_v)
    pltpu.sync_copy(x_hbm.at[pl.ds(base, BPW)], idx_v)
    pltpu.sync_copy(cond_hbm.at[pl.ds(base, BPW)], ci_v)
    pltpu.async_copy(ct_hbm.at[ci_v], cr_v, csem).wait()

    start_gather(0, 0)
    start_gather(1, 1)

    @pl.loop(0, BPW // R)
    def _(i):
        j0 = R * i
        for p in range(R):
            j = j0 + p
            buf = p
            nbuf = (p + 2) % R

            @pl.when(j + 2 < BPW)
            def _():
                @pl.when(j - 2 >= 0)
                def _():
                    out_desc(nbuf, j - 2).wait()
                start_gather(nbuf, j + 2)

            wait_gather(buf, j)
            compute(buf, j)
            out_desc(buf, j).start()

    out_desc(0, BPW - 4).wait()
    out_desc(1, BPW - 3).wait()
    out_desc(2, BPW - 2).wait()
    out_desc(3, BPW - 1).wait()


def kernel(x, cond, latent_table, cond_table, pe):
    return _embed(x, cond, latent_table, cond_table, pe[:SEQ])
